# baseline (speedup 1.0000x reference)
import functools

import jax
import jax.numpy as jnp
from jax import lax
from jax.experimental import pallas as pl
from jax.experimental.pallas import tpu as pltpu
from jax.experimental.pallas import tpu_sc as plsc


def _top2(probs, tile, ne):
    eidx = jax.lax.broadcasted_iota(jnp.int32, (tile, ne), 1)
    p0 = jnp.max(probs, axis=-1, keepdims=True)
    e0 = jnp.min(jnp.where(probs == p0, eidx, ne), axis=-1, keepdims=True)
    masked = jnp.where(eidx == e0, -jnp.inf, probs)
    p1 = jnp.max(masked, axis=-1, keepdims=True)
    e1 = jnp.min(jnp.where(masked == p1, eidx, ne), axis=-1, keepdims=True)
    return p0, e0, p1, e1


def _tc_router(x_ref, w1_ref, b1_ref, w2_ref, b2_ref,
               disp_ref, comb_ref, probs_ref, aux_ref, tbl_ref, c0f_ref,
               c0_s, sums_s, *, tile, tiles, ne, cap, ntok):
    t = pl.program_id(0)
    eidx = jax.lax.broadcasted_iota(jnp.int32, (tile, ne), 1)

    disp_ref[...] = jnp.zeros((tile, ne, cap), jnp.float32)
    comb_ref[...] = jnp.zeros((tile, ne, cap), jnp.float32)

    x = x_ref[...]
    h = jnp.maximum(
        jnp.dot(x, w1_ref[...], preferred_element_type=jnp.float32)
        + b1_ref[...], 0.0)
    logits = (jnp.dot(h, w2_ref[...], preferred_element_type=jnp.float32)
              + b2_ref[...])
    m = jnp.max(logits, axis=-1, keepdims=True)
    ex = jnp.exp(logits - m)
    probs = ex / jnp.sum(ex, axis=-1, keepdims=True)
    probs_ref[...] = probs

    p0, e0, p1, e1 = _top2(probs, tile, ne)
    s = p0 + p1
    p0n = p0 / s
    p1n = p1 / s
    tbl_ref[...] = jnp.concatenate(
        [jnp.reshape(e0.astype(jnp.float32), (1, tile)),
         jnp.reshape(e1.astype(jnp.float32), (1, tile)),
         jnp.reshape(p0n, (1, tile)),
         jnp.reshape(p1n, (1, tile)),
         jnp.zeros((4, tile), jnp.float32)], axis=0)

    flags = jnp.max((eidx == e0).astype(jnp.float32), axis=0, keepdims=True)
    psum = jnp.sum(probs, axis=0, keepdims=True)
    first = t == 0
    c0 = jnp.where(first, flags, jnp.maximum(c0_s[...], flags))
    c0_s[...] = c0
    sums_s[...] = jnp.where(first, psum, sums_s[...] + psum)

    @pl.when(t == tiles - 1)
    def _tail():
        mean = sums_s[...] * (1.0 / ntok)
        aux_ref[...] = jnp.sum(mean * jnp.log(mean * ne + 1e-09),
                               axis=-1, keepdims=True)
        c0f_ref[...] = jnp.concatenate(
            [c0, jnp.zeros((1, 16 - ne), jnp.float32)], axis=1)


def _make_sc_scatter(ntok, ne, cap):
    info = plsc.get_sparse_core_info()
    nw = info.num_cores * info.num_subcores
    nc = info.num_cores
    bpw = ntok // nw
    mesh = plsc.VectorSubcoreMesh(core_axis_name="c", subcore_axis_name="s")

    @functools.partial(
        pl.kernel, mesh=mesh, out_type=(),
        compiler_params=pltpu.CompilerParams(needs_layout_passes=False),
        scratch_types=[
            pltpu.VMEM((bpw,), jnp.float32),
            pltpu.VMEM((bpw,), jnp.float32),
            pltpu.VMEM((bpw,), jnp.float32),
            pltpu.VMEM((bpw,), jnp.float32),
            pltpu.VMEM((16,), jnp.float32),
            pltpu.VMEM((2 * bpw,), jnp.int32),
            pltpu.VMEM((2 * bpw,), jnp.float32),
            pltpu.VMEM((2 * bpw,), jnp.float32),
            pltpu.SemaphoreType.DMA,
            pltpu.SemaphoreType.DMA,
        ],
    )
    def _sc_scatter(tbl_hbm, c0_hbm, disp_ref, comb_ref,
                    e0_v, e1_v, p0_v, p1_v, c0_v, idx_v, vd_v, vc_v,
                    semd, semc):
        wid = lax.axis_index("s") * nc + lax.axis_index("c")
        base = wid * bpw
        pltpu.sync_copy(tbl_hbm.at[0, pl.ds(base, bpw)], e0_v)
        pltpu.sync_copy(tbl_hbm.at[1, pl.ds(base, bpw)], e1_v)
        pltpu.sync_copy(tbl_hbm.at[2, pl.ds(base, bpw)], p0_v)
        pltpu.sync_copy(tbl_hbm.at[3, pl.ds(base, bpw)], p1_v)
        pltpu.sync_copy(c0_hbm, c0_v)
        lanes = lax.iota(jnp.int32, 16)
        c0vec = c0_v[...]
        c0e = [jnp.sum(jnp.where(lanes == e, c0vec, 0.0)) for e in range(ne)]
        for k in range(bpw // 16):
            sl = pl.ds(k * 16, 16)
            sl2 = pl.ds(bpw + k * 16, 16)
            e0 = e0_v[sl].astype(jnp.int32)
            e1i = e1_v[sl]
            e1 = e1i.astype(jnp.int32)
            p0 = p0_v[sl]
            p1 = p1_v[sl]
            slot1f = jnp.zeros((16,), jnp.float32)
            for e in range(ne):
                slot1f = slot1f + jnp.where(e1i == float(e), c0e[e], 0.0)
            slot1 = slot1f.astype(jnp.int32)
            nbase = (lanes + (base + k * 16)) * (ne * cap)
            idx_v[sl] = nbase + e0 * cap
            idx_v[sl2] = nbase + e1 * cap + slot1
            ones = jnp.full((16,), 1.0, jnp.float32)
            vd_v[sl] = ones
            vd_v[sl2] = ones
            vc_v[sl] = p0
            vc_v[sl2] = p1
        d = pltpu.async_copy(vd_v, disp_ref.at[idx_v], semd)
        c = pltpu.async_copy(vc_v, comb_ref.at[idx_v], semc)
        d.wait()
        c.wait()

    return _sc_scatter


def kernel(hidden_states, W1, b1, W2, b2):
    bb, ss, hh = hidden_states.shape
    ne = W2.shape[1]
    ntok = bb * ss
    cap = int(bb * ss * 1.5 * 2 / ne)
    x = hidden_states.reshape(ntok, hh)
    b1r = b1.reshape(1, hh)
    b2r = b2.reshape(1, ne)
    tile = 256
    tiles = ntok // tile

    body = functools.partial(_tc_router, tile=tile, tiles=tiles,
                             ne=ne, cap=cap, ntok=ntok)

    disp_z, comb_z, probs, aux, tbl, c0f = pl.pallas_call(
        body,
        grid=(tiles,),
        in_specs=[
            pl.BlockSpec((tile, hh), lambda t: (t, 0)),
            pl.BlockSpec((hh, hh), lambda t: (0, 0)),
            pl.BlockSpec((1, hh), lambda t: (0, 0)),
            pl.BlockSpec((hh, ne), lambda t: (0, 0)),
            pl.BlockSpec((1, ne), lambda t: (0, 0)),
        ],
        out_specs=[
            pl.BlockSpec((tile, ne, cap), lambda t: (t, 0, 0)),
            pl.BlockSpec((tile, ne, cap), lambda t: (t, 0, 0)),
            pl.BlockSpec((tile, ne), lambda t: (t, 0)),
            pl.BlockSpec((1, 1), lambda t: (0, 0)),
            pl.BlockSpec((8, tile), lambda t: (0, t)),
            pl.BlockSpec((1, 16), lambda t: (0, 0)),
        ],
        out_shape=[
            jax.ShapeDtypeStruct((ntok, ne, cap), jnp.float32),
            jax.ShapeDtypeStruct((ntok, ne, cap), jnp.float32),
            jax.ShapeDtypeStruct((ntok, ne), jnp.float32),
            jax.ShapeDtypeStruct((1, 1), jnp.float32),
            jax.ShapeDtypeStruct((8, ntok), jnp.float32),
            jax.ShapeDtypeStruct((1, 16), jnp.float32),
        ],
        scratch_shapes=[
            pltpu.VMEM((1, ne), jnp.float32),
            pltpu.VMEM((1, ne), jnp.float32),
        ],
        compiler_params=pltpu.CompilerParams(
            dimension_semantics=("arbitrary",)),
    )(x, W1, b1r, W2, b2r)

    sc_scatter = _make_sc_scatter(ntok, ne, cap)
    dref = jax.new_ref(disp_z.reshape(ntok * ne * cap))
    cref = jax.new_ref(comb_z.reshape(ntok * ne * cap))
    sc_scatter(tbl, c0f.reshape(16), dref, cref)
    disp = dref[...].reshape(bb, ss, ne, cap)
    comb = cref[...].reshape(bb, ss, ne, cap)

    return (disp, comb, probs.reshape(bb, ss, ne), aux.reshape(()))


# speedup vs baseline: 1.5841x; 1.5841x over previous
import functools

import jax
import jax.numpy as jnp
from jax import lax
from jax.experimental import pallas as pl
from jax.experimental.pallas import tpu as pltpu
from jax.experimental.pallas import tpu_sc as plsc


def _top2(probs, tile, ne):
    eidx = jax.lax.broadcasted_iota(jnp.int32, (tile, ne), 1)
    p0 = jnp.max(probs, axis=-1, keepdims=True)
    e0 = jnp.min(jnp.where(probs == p0, eidx, ne), axis=-1, keepdims=True)
    masked = jnp.where(eidx == e0, -jnp.inf, probs)
    p1 = jnp.max(masked, axis=-1, keepdims=True)
    e1 = jnp.min(jnp.where(masked == p1, eidx, ne), axis=-1, keepdims=True)
    return p0, e0, p1, e1


def _tc_router(x_ref, w1_ref, b1_ref, w2_ref, b2_ref,
               disp_ref, comb_ref, probs_ref, aux_ref, tbl_ref, c0f_ref,
               c0_s, sums_s, *, tile, tiles, ne, cap, ntok):
    t = pl.program_id(0)
    eidx = jax.lax.broadcasted_iota(jnp.int32, (tile, ne), 1)

    disp_ref[...] = jnp.zeros((tile * ne * cap,), jnp.float32)
    comb_ref[...] = jnp.zeros((tile * ne * cap,), jnp.float32)

    x = x_ref[...]
    h = jnp.maximum(
        jnp.dot(x, w1_ref[...], preferred_element_type=jnp.float32)
        + b1_ref[...], 0.0)
    logits = (jnp.dot(h, w2_ref[...], preferred_element_type=jnp.float32)
              + b2_ref[...])
    m = jnp.max(logits, axis=-1, keepdims=True)
    ex = jnp.exp(logits - m)
    probs = ex / jnp.sum(ex, axis=-1, keepdims=True)
    probs_ref[...] = probs

    p0, e0, p1, e1 = _top2(probs, tile, ne)
    s = p0 + p1
    p0n = p0 / s
    p1n = p1 / s
    tbl_ref[...] = jnp.concatenate(
        [jnp.reshape(e0.astype(jnp.float32), (1, tile)),
         jnp.reshape(e1.astype(jnp.float32), (1, tile)),
         jnp.reshape(p0n, (1, tile)),
         jnp.reshape(p1n, (1, tile)),
         jnp.zeros((4, tile), jnp.float32)], axis=0)

    flags = jnp.max((eidx == e0).astype(jnp.float32), axis=0, keepdims=True)
    psum = jnp.sum(probs, axis=0, keepdims=True)
    first = t == 0
    c0 = jnp.where(first, flags, jnp.maximum(c0_s[...], flags))
    c0_s[...] = c0
    sums_s[...] = jnp.where(first, psum, sums_s[...] + psum)

    @pl.when(t == tiles - 1)
    def _tail():
        mean = sums_s[...] * (1.0 / ntok)
        aux_ref[...] = jnp.sum(mean * jnp.log(mean * ne + 1e-09),
                               axis=-1, keepdims=True)
        c0f_ref[...] = jnp.concatenate(
            [c0, jnp.zeros((1, 16 - ne), jnp.float32)], axis=1)


def _make_sc_scatter(ntok, ne, cap):
    info = plsc.get_sparse_core_info()
    nw = info.num_cores * info.num_subcores
    nc = info.num_cores
    bpw = ntok // nw
    mesh = plsc.VectorSubcoreMesh(core_axis_name="c", subcore_axis_name="s")

    @functools.partial(
        pl.kernel, mesh=mesh, out_type=(),
        compiler_params=pltpu.CompilerParams(needs_layout_passes=False),
        scratch_types=[
            pltpu.VMEM((bpw,), jnp.float32),
            pltpu.VMEM((bpw,), jnp.float32),
            pltpu.VMEM((bpw,), jnp.float32),
            pltpu.VMEM((bpw,), jnp.float32),
            pltpu.VMEM((16,), jnp.float32),
            pltpu.VMEM((2 * bpw,), jnp.int32),
            pltpu.VMEM((2 * bpw,), jnp.float32),
            pltpu.VMEM((2 * bpw,), jnp.float32),
            pltpu.SemaphoreType.DMA,
            pltpu.SemaphoreType.DMA,
        ],
    )
    def _sc_scatter(tbl_hbm, c0_hbm, disp_ref, comb_ref,
                    e0_v, e1_v, p0_v, p1_v, c0_v, idx_v, vd_v, vc_v,
                    semd, semc):
        wid = lax.axis_index("s") * nc + lax.axis_index("c")
        base = wid * bpw
        pltpu.sync_copy(tbl_hbm.at[0, pl.ds(base, bpw)], e0_v)
        pltpu.sync_copy(tbl_hbm.at[1, pl.ds(base, bpw)], e1_v)
        pltpu.sync_copy(tbl_hbm.at[2, pl.ds(base, bpw)], p0_v)
        pltpu.sync_copy(tbl_hbm.at[3, pl.ds(base, bpw)], p1_v)
        pltpu.sync_copy(c0_hbm, c0_v)
        lanes = lax.iota(jnp.int32, 16)
        c0vec = c0_v[...]
        c0e = [jnp.sum(jnp.where(lanes == e, c0vec, 0.0)) for e in range(ne)]
        for k in range(bpw // 16):
            sl = pl.ds(k * 16, 16)
            sl2 = pl.ds(bpw + k * 16, 16)
            e0 = e0_v[sl].astype(jnp.int32)
            e1i = e1_v[sl]
            e1 = e1i.astype(jnp.int32)
            p0 = p0_v[sl]
            p1 = p1_v[sl]
            slot1f = jnp.zeros((16,), jnp.float32)
            for e in range(ne):
                slot1f = slot1f + jnp.where(e1i == float(e), c0e[e], 0.0)
            slot1 = slot1f.astype(jnp.int32)
            nbase = (lanes + (base + k * 16)) * (ne * cap)
            idx_v[sl] = nbase + e0 * cap
            idx_v[sl2] = nbase + e1 * cap + slot1
            ones = jnp.full((16,), 1.0, jnp.float32)
            vd_v[sl] = ones
            vd_v[sl2] = ones
            vc_v[sl] = p0
            vc_v[sl2] = p1
        d = pltpu.async_copy(vd_v, disp_ref.at[idx_v], semd)
        c = pltpu.async_copy(vc_v, comb_ref.at[idx_v], semc)
        d.wait()
        c.wait()

    return _sc_scatter


def kernel(hidden_states, W1, b1, W2, b2):
    bb, ss, hh = hidden_states.shape
    ne = W2.shape[1]
    ntok = bb * ss
    cap = int(bb * ss * 1.5 * 2 / ne)
    x = hidden_states.reshape(ntok, hh)
    b1r = b1.reshape(1, hh)
    b2r = b2.reshape(1, ne)
    tile = 256
    tiles = ntok // tile

    body = functools.partial(_tc_router, tile=tile, tiles=tiles,
                             ne=ne, cap=cap, ntok=ntok)

    disp_z, comb_z, probs, aux, tbl, c0f = pl.pallas_call(
        body,
        grid=(tiles,),
        in_specs=[
            pl.BlockSpec((tile, hh), lambda t: (t, 0)),
            pl.BlockSpec((hh, hh), lambda t: (0, 0)),
            pl.BlockSpec((1, hh), lambda t: (0, 0)),
            pl.BlockSpec((hh, ne), lambda t: (0, 0)),
            pl.BlockSpec((1, ne), lambda t: (0, 0)),
        ],
        out_specs=[
            pl.BlockSpec((tile * ne * cap,), lambda t: (t,)),
            pl.BlockSpec((tile * ne * cap,), lambda t: (t,)),
            pl.BlockSpec((tile, ne), lambda t: (t, 0)),
            pl.BlockSpec((1, 1), lambda t: (0, 0)),
            pl.BlockSpec((8, tile), lambda t: (0, t)),
            pl.BlockSpec((1, 16), lambda t: (0, 0)),
        ],
        out_shape=[
            jax.ShapeDtypeStruct((ntok * ne * cap,), jnp.float32),
            jax.ShapeDtypeStruct((ntok * ne * cap,), jnp.float32),
            jax.ShapeDtypeStruct((ntok, ne), jnp.float32),
            jax.ShapeDtypeStruct((1, 1), jnp.float32),
            jax.ShapeDtypeStruct((8, ntok), jnp.float32),
            jax.ShapeDtypeStruct((1, 16), jnp.float32),
        ],
        scratch_shapes=[
            pltpu.VMEM((1, ne), jnp.float32),
            pltpu.VMEM((1, ne), jnp.float32),
        ],
        compiler_params=pltpu.CompilerParams(
            dimension_semantics=("arbitrary",)),
    )(x, W1, b1r, W2, b2r)

    sc_scatter = _make_sc_scatter(ntok, ne, cap)
    dref = jax.new_ref(disp_z)
    cref = jax.new_ref(comb_z)
    sc_scatter(tbl, c0f.reshape(16), dref, cref)
    disp = dref[...].reshape(bb, ss, ne, cap)
    comb = cref[...].reshape(bb, ss, ne, cap)

    return (disp, comb, probs.reshape(bb, ss, ne), aux.reshape(()))


# speedup vs baseline: 6.0042x; 3.7903x over previous
import functools

import jax
import jax.numpy as jnp
from jax.experimental import pallas as pl
from jax.experimental.pallas import tpu as pltpu

_LEAD = 128


def _top2(probs, tile, ne):
    eidx = jax.lax.broadcasted_iota(jnp.int32, (tile, ne), 1)
    p0 = jnp.max(probs, axis=-1, keepdims=True)
    e0 = jnp.min(jnp.where(probs == p0, eidx, ne), axis=-1, keepdims=True)
    masked = jnp.where(eidx == e0, -jnp.inf, probs)
    p1 = jnp.max(masked, axis=-1, keepdims=True)
    e1 = jnp.min(jnp.where(masked == p1, eidx, ne), axis=-1, keepdims=True)
    return p0, e0, p1, e1


def _router_kernel(x_ref, w1_ref, b1_ref, w2_ref, b2_ref,
                   disp_ref, comb_ref, probs_ref, aux_ref,
                   probs_s, c0_s, sums_s, zbuf, dbuf, cbuf, sem_z, sem_c,
                   *, tile, tiles, ne, cap, ntok):
    p = pl.program_id(0)
    t = pl.program_id(1)
    eidx = jax.lax.broadcasted_iota(jnp.int32, (tile, ne), 1)
    ztail = cap - _LEAD

    def _zcopy(dst_ref, row, qi):
        return pltpu.make_async_copy(
            zbuf, dst_ref.at[pl.ds(row, tile), :, pl.ds(_LEAD, ztail)],
            sem_z.at[qi])

    @pl.when(p == 0)
    def _pass0():
        @pl.when(t == 0)
        def _init():
            zbuf[...] = jnp.zeros_like(zbuf)

        _zcopy(disp_ref, t * tile, (2 * t) % 4).start()
        _zcopy(comb_ref, t * tile, (2 * t + 1) % 4).start()

        x = x_ref[...]
        h = jnp.maximum(
            jnp.dot(x, w1_ref[...], preferred_element_type=jnp.float32)
            + b1_ref[...], 0.0)
        logits = (jnp.dot(h, w2_ref[...], preferred_element_type=jnp.float32)
                  + b2_ref[...])
        m = jnp.max(logits, axis=-1, keepdims=True)
        ex = jnp.exp(logits - m)
        probs = ex / jnp.sum(ex, axis=-1, keepdims=True)
        probs_s[pl.ds(t * tile, tile), :] = probs

        _, e0, _, _ = _top2(probs, tile, ne)
        flags = jnp.max((eidx == e0).astype(jnp.float32), axis=0,
                        keepdims=True)
        psum = jnp.sum(probs, axis=0, keepdims=True)
        first = t == 0
        c0_s[...] = jnp.where(first, flags, jnp.maximum(c0_s[...], flags))
        sums_s[...] = jnp.where(first, psum, sums_s[...] + psum)

        @pl.when(t == tiles - 1)
        def _aux():
            mean = sums_s[...] * (1.0 / ntok)
            aux_ref[...] = jnp.sum(mean * jnp.log(mean * ne + 1e-09),
                                   axis=-1, keepdims=True)

    @pl.when(p == 1)
    def _pass1():
        probs = probs_s[pl.ds(t * tile, tile), :]
        p0, e0, p1, e1 = _top2(probs, tile, ne)
        s = p0 + p1
        p0n = p0 / s
        p1n = p1 / s
        c0 = c0_s[...]
        slot1 = jnp.sum(jnp.where(eidx == e1, c0, 0.0), axis=-1,
                        keepdims=True).astype(jnp.int32)
        slotmat = jnp.where(eidx == e0, 0,
                            jnp.where(eidx == e1, slot1, -1))
        valmat = jnp.where(eidx == e0, p0n,
                           jnp.where(eidx == e1, p1n, 0.0))
        slot = t % 2

        def _ccopy(src, dst_ref, s, step):
            return pltpu.make_async_copy(
                src.at[s],
                dst_ref.at[pl.ds(step * tile, tile), :, pl.ds(0, _LEAD)],
                sem_c.at[s])

        @pl.when(t >= 2)
        def _reuse_wait():
            _ccopy(dbuf, disp_ref, slot, t - 2).wait()
            _ccopy(cbuf, comb_ref, slot, t - 2).wait()

        siota = jax.lax.broadcasted_iota(jnp.int32, (tile, ne, _LEAD), 2)
        hit = siota == slotmat[:, :, None]
        dbuf[slot] = hit.astype(jnp.float32)
        cbuf[slot] = jnp.where(hit, valmat[:, :, None], 0.0)
        _ccopy(dbuf, disp_ref, slot, t).start()
        _ccopy(cbuf, comb_ref, slot, t).start()
        probs_ref[...] = probs

        @pl.when(t == tiles - 1)
        def _drain():
            _ccopy(dbuf, disp_ref, 1 - slot, t - 1).wait()
            _ccopy(cbuf, comb_ref, 1 - slot, t - 1).wait()
            _ccopy(dbuf, disp_ref, slot, t).wait()
            _ccopy(cbuf, comb_ref, slot, t).wait()
            for qi in range(4):
                for _ in range(2 * tiles // 4):
                    _zcopy(disp_ref, 0, qi).wait()


def kernel(hidden_states, W1, b1, W2, b2):
    bb, ss, hh = hidden_states.shape
    ne = W2.shape[1]
    ntok = bb * ss
    cap = int(bb * ss * 1.5 * 2 / ne)
    x = hidden_states.reshape(ntok, hh)
    b1r = b1.reshape(1, hh)
    b2r = b2.reshape(1, ne)
    tile = 512
    tiles = ntok // tile

    body = functools.partial(_router_kernel, tile=tile, tiles=tiles,
                             ne=ne, cap=cap, ntok=ntok)

    disp, comb, probs, aux = pl.pallas_call(
        body,
        grid=(2, tiles),
        in_specs=[
            pl.BlockSpec((tile, hh), lambda p, t: (jnp.where(p == 0, t, 0), 0)),
            pl.BlockSpec((hh, hh), lambda p, t: (0, 0)),
            pl.BlockSpec((1, hh), lambda p, t: (0, 0)),
            pl.BlockSpec((hh, ne), lambda p, t: (0, 0)),
            pl.BlockSpec((1, ne), lambda p, t: (0, 0)),
        ],
        out_specs=[
            pl.BlockSpec(memory_space=pl.ANY),
            pl.BlockSpec(memory_space=pl.ANY),
            pl.BlockSpec((tile, ne), lambda p, t: (jnp.where(p == 1, t, 0), 0)),
            pl.BlockSpec((1, 1), lambda p, t: (0, 0)),
        ],
        out_shape=[
            jax.ShapeDtypeStruct((ntok, ne, cap), jnp.float32),
            jax.ShapeDtypeStruct((ntok, ne, cap), jnp.float32),
            jax.ShapeDtypeStruct((ntok, ne), jnp.float32),
            jax.ShapeDtypeStruct((1, 1), jnp.float32),
        ],
        scratch_shapes=[
            pltpu.VMEM((ntok, ne), jnp.float32),
            pltpu.VMEM((1, ne), jnp.float32),
            pltpu.VMEM((1, ne), jnp.float32),
            pltpu.VMEM((tile, ne, cap - _LEAD), jnp.float32),
            pltpu.VMEM((2, tile, ne, _LEAD), jnp.float32),
            pltpu.VMEM((2, tile, ne, _LEAD), jnp.float32),
            pltpu.SemaphoreType.DMA((4,)),
            pltpu.SemaphoreType.DMA((2,)),
        ],
        compiler_params=pltpu.CompilerParams(
            dimension_semantics=("arbitrary", "arbitrary")),
    )(x, W1, b1r, W2, b2r)

    return (disp.reshape(bb, ss, ne, cap),
            comb.reshape(bb, ss, ne, cap),
            probs.reshape(bb, ss, ne),
            aux.reshape(()))
